# Initial kernel scaffold; baseline (speedup 1.0000x reference)
import jax
import jax.numpy as jnp
from jax.experimental import pallas as pl


def kernel(weight, scale):
    raise NotImplementedError("write your pallas kernel here")


# speedup vs baseline: 2692.3758x; 2692.3758x over previous
import jax
import jax.numpy as jnp
from jax.experimental import pallas as pl

_Q = 15.0
_STD_DEV_NUM = 2.0
_COEFFICIENT = 0.001
_ROWS_PER_BLOCK = 256


def _loss_block(w_ref, s_ref, out_ref):
    i = pl.program_id(0)
    w = w_ref[...]
    s = s_ref[...]
    k = w.shape[1]
    mean = jnp.mean(w, axis=1, keepdims=True)
    var = jnp.sum((w - mean) ** 2, axis=1, keepdims=True) / (k - 1)
    std = jnp.sqrt(var)
    lower = mean - _STD_DEV_NUM * std
    step = (2.0 * _STD_DEV_NUM / _Q) * std
    x = (w - lower) / step
    idx = jnp.floor(jnp.clip(x, 0.0, _Q - 1.0))
    target = s * (idx - 7.0)
    partial = jnp.sum(jnp.abs(w - target)).reshape(1, 1)

    @pl.when(i == 0)
    def _init():
        out_ref[...] = jnp.zeros_like(out_ref)

    out_ref[...] += partial


def kernel(weight, scale):
    c, k = weight.shape
    r = _ROWS_PER_BLOCK
    out = pl.pallas_call(
        _loss_block,
        grid=(c // r,),
        in_specs=[
            pl.BlockSpec((r, k), lambda i: (i, 0)),
            pl.BlockSpec((r, 1), lambda i: (i, 0)),
        ],
        out_specs=pl.BlockSpec((1, 1), lambda i: (0, 0)),
        out_shape=jax.ShapeDtypeStruct((1, 1), jnp.float32),
    )(weight, scale)
    return out[0, 0] * _COEFFICIENT
